# baseline (speedup 1.0000x reference)
import jax
import jax.numpy as jnp
from jax.experimental import pallas as pl

B = 32
A = 8732
C = 85
AB = 624
NBLK = 14
APAD = NBLK * AB
NEG_FACTOR_K = 3
I32_MIN = -(2 ** 31)
I32_MAX = 2 ** 31 - 1


def _stage_a_body(pos_ref, pred_ref, gts_ref,
                  keys_ref, ce_ref, loc_ref, posce_ref, n_ref):
    b = pl.program_id(0)
    a = pl.program_id(1)
    x = pred_ref[0]
    g = gts_ref[0]
    p = pos_ref[0]

    lane = jax.lax.broadcasted_iota(jnp.int32, (AB, C), 1)
    sub = jax.lax.broadcasted_iota(jnp.int32, (AB, 1), 0)
    valid = (a * AB + sub) < A
    posb = (p > 0) & valid

    d = x - g
    ad = jnp.abs(d)
    sl1 = jnp.where(lane < 4, jnp.where(ad < 1.0, 0.5 * d * d, ad - 0.5), 0.0)
    locv = jnp.sum(sl1, axis=1, keepdims=True)

    isc = lane >= 4
    xm = jnp.where(isc, x, -jnp.inf)
    m = jnp.max(xm, axis=1, keepdims=True)
    e = jnp.where(isc, jnp.exp(x - m), 0.0)
    s = jnp.sum(e, axis=1, keepdims=True)
    lse = jnp.log(s) + m
    bg = jnp.sum(jnp.where(lane == C - 1, x, 0.0), axis=1, keepdims=True)
    bl = lse - bg

    gm = jnp.where(isc, g, -jnp.inf)
    gmax = jnp.max(gm, axis=1, keepdims=True)
    lab = jnp.min(jnp.where(isc & (g == gmax), lane, 2 * C),
                  axis=1, keepdims=True)
    pa = jnp.sum(jnp.where(lane == lab, x, 0.0), axis=1, keepdims=True)
    ce = lse - pa

    kbits = jax.lax.bitcast_convert_type(bl, jnp.int32)
    key = jnp.where(posb, I32_MIN, kbits)
    key = jnp.where(valid, key, -1)

    keys_ref[0] = key
    ce_ref[0] = ce

    @pl.when((b == 0) & (a == 0))
    def _():
        loc_ref[...] = jnp.zeros_like(loc_ref)
        posce_ref[...] = jnp.zeros_like(posce_ref)
        n_ref[...] = jnp.zeros_like(n_ref)

    loc_ref[...] += jnp.sum(jnp.where(posb, locv, 0.0))
    posce_ref[...] += jnp.sum(jnp.where(posb, ce, 0.0))
    n_ref[...] += jnp.sum(jnp.where(posb, 1.0, 0.0))


def _stage_b_body(keys_ref, ce_ref, neg_ref):
    keys = keys_ref[...]
    ce = ce_ref[...]

    posm = keys == I32_MIN
    pos_num = jnp.sum(posm.astype(jnp.int32), axis=1, keepdims=True)
    k = jnp.minimum(A - pos_num, NEG_FACTOR_K * pos_num)

    lo = jnp.zeros_like(k)
    hi = jnp.full_like(k, I32_MAX - 1)

    def body(_, lohi):
        lo, hi = lohi
        mid = lo + ((hi - lo + 1) >> 1)
        cnt = jnp.sum((keys >= mid).astype(jnp.int32), axis=1, keepdims=True)
        ge = cnt >= k
        return (jnp.where(ge, mid, lo), jnp.where(ge, hi, mid - 1))

    lo, _ = jax.lax.fori_loop(0, 31, body, (lo, hi))
    vstar = lo

    gt = keys > vstar
    c_gt = jnp.sum(gt.astype(jnp.int32), axis=1, keepdims=True)
    need = k - c_gt
    eq = keys == vstar
    lane = jax.lax.broadcasted_iota(jnp.int32, (B, APAD), 1)

    lo2 = jnp.zeros_like(k)
    hi2 = jnp.full_like(k, APAD)

    def body2(_, lohi):
        lo2, hi2 = lohi
        mid = (lo2 + hi2) >> 1
        gcnt = jnp.sum((eq & (lane < mid)).astype(jnp.int32),
                       axis=1, keepdims=True)
        geq = gcnt >= need
        return (jnp.where(geq, lo2, mid + 1), jnp.where(geq, mid, hi2))

    tstar, _ = jax.lax.fori_loop(0, 14, body2, (lo2, hi2))

    sel = gt | (eq & (lane < tstar))
    neg_ref[...] = jnp.zeros_like(neg_ref) + jnp.sum(jnp.where(sel, ce, 0.0))


def kernel(pos_indicator, predicts, gts):
    pos = pos_indicator.astype(jnp.int32)
    pos = jnp.pad(pos, ((0, 0), (0, APAD - A)))
    posr = pos.reshape(B * NBLK, AB, 1)

    keys, cev, locsum, posce, n = pl.pallas_call(
        _stage_a_body,
        grid=(B, NBLK),
        in_specs=[
            pl.BlockSpec((1, AB, 1), lambda b, a: (b * NBLK + a, 0, 0)),
            pl.BlockSpec((1, AB, C), lambda b, a: (b, a, 0)),
            pl.BlockSpec((1, AB, C), lambda b, a: (b, a, 0)),
        ],
        out_specs=[
            pl.BlockSpec((1, AB, 1), lambda b, a: (b * NBLK + a, 0, 0)),
            pl.BlockSpec((1, AB, 1), lambda b, a: (b * NBLK + a, 0, 0)),
            pl.BlockSpec((1, 1), lambda b, a: (0, 0)),
            pl.BlockSpec((1, 1), lambda b, a: (0, 0)),
            pl.BlockSpec((1, 1), lambda b, a: (0, 0)),
        ],
        out_shape=[
            jax.ShapeDtypeStruct((B * NBLK, AB, 1), jnp.int32),
            jax.ShapeDtypeStruct((B * NBLK, AB, 1), jnp.float32),
            jax.ShapeDtypeStruct((1, 1), jnp.float32),
            jax.ShapeDtypeStruct((1, 1), jnp.float32),
            jax.ShapeDtypeStruct((1, 1), jnp.float32),
        ],
    )(posr, predicts, gts)

    keys2 = keys.reshape(B, APAD)
    ce2 = cev.reshape(B, APAD)

    negsum = pl.pallas_call(
        _stage_b_body,
        in_specs=[
            pl.BlockSpec((B, APAD), lambda: (0, 0)),
            pl.BlockSpec((B, APAD), lambda: (0, 0)),
        ],
        out_specs=pl.BlockSpec((1, 1), lambda: (0, 0)),
        out_shape=jax.ShapeDtypeStruct((1, 1), jnp.float32),
    )(keys2, ce2)

    nn = n[0, 0]
    conf_loss = (posce[0, 0] + negsum[0, 0]) / nn
    loc_loss = locsum[0, 0] / nn
    return (conf_loss, loc_loss)


# speedup vs baseline: 1.8444x; 1.8444x over previous
import jax
import jax.numpy as jnp
from jax.experimental import pallas as pl

B = 32
A = 8732
C = 85
AB = 624
NBLK = 14
APAD = NBLK * AB
NEG_FACTOR_K = 3
I32_MIN = -(2 ** 31)
I32_MAX = 2 ** 31 - 1


def _stage_a_body(pos_ref, pred_ref, gts_ref,
                  keys_ref, ce_ref, loc_ref, posce_ref, n_ref):
    b = pl.program_id(0)
    a = pl.program_id(1)
    xT = pred_ref[0].T
    gT = gts_ref[0].T
    p = pos_ref[0]

    sub = jax.lax.broadcasted_iota(jnp.int32, (C, AB), 0)
    lanev = jax.lax.broadcasted_iota(jnp.int32, (1, AB), 1)
    valid = (a * AB + lanev) < A
    posb = (p > 0) & valid

    d = xT[0:8, :] - gT[0:8, :]
    ad = jnp.abs(d)
    sl1 = jnp.where(sub[0:8, :] < 4,
                    jnp.where(ad < 1.0, 0.5 * d * d, ad - 0.5), 0.0)
    locv = jnp.sum(sl1, axis=0, keepdims=True)

    isc = sub >= 4
    xm = jnp.where(isc, xT, -jnp.inf)
    m = jnp.max(xm, axis=0, keepdims=True)
    e = jnp.exp(xm - m)
    s = jnp.sum(e, axis=0, keepdims=True)
    lse = jnp.log(s) + m
    bg = xT[C - 1:C, :]
    bl = lse - bg

    gm = jnp.where(isc, gT, -jnp.inf)
    gmax = jnp.max(gm, axis=0, keepdims=True)
    lab = jnp.min(jnp.where(gm == gmax, sub, 2 * C),
                  axis=0, keepdims=True)
    pa = jnp.sum(jnp.where(sub == lab, xT, 0.0), axis=0, keepdims=True)
    ce = lse - pa

    kbits = jax.lax.bitcast_convert_type(bl, jnp.int32)
    key = jnp.where(posb, I32_MIN, kbits)
    key = jnp.where(valid, key, -1)

    keys_ref[0] = key
    ce_ref[0] = ce

    @pl.when((b == 0) & (a == 0))
    def _():
        loc_ref[...] = jnp.zeros_like(loc_ref)
        posce_ref[...] = jnp.zeros_like(posce_ref)
        n_ref[...] = jnp.zeros_like(n_ref)

    loc_ref[...] += jnp.sum(jnp.where(posb, locv, 0.0))
    posce_ref[...] += jnp.sum(jnp.where(posb, ce, 0.0))
    n_ref[...] += jnp.sum(jnp.where(posb, 1.0, 0.0))


def _stage_b_body(keys_ref, ce_ref, neg_ref):
    keys = keys_ref[...]
    ce = ce_ref[...]

    posm = keys == I32_MIN
    pos_num = jnp.sum(posm.astype(jnp.int32), axis=1, keepdims=True)
    k = jnp.minimum(A - pos_num, NEG_FACTOR_K * pos_num)

    lo = jnp.zeros_like(k)
    hi = jnp.full_like(k, I32_MAX - 1)

    def body(_, lohi):
        lo, hi = lohi
        mid = lo + ((hi - lo + 1) >> 1)
        cnt = jnp.sum((keys >= mid).astype(jnp.int32), axis=1, keepdims=True)
        ge = cnt >= k
        return (jnp.where(ge, mid, lo), jnp.where(ge, hi, mid - 1))

    lo, _ = jax.lax.fori_loop(0, 31, body, (lo, hi))
    vstar = lo

    gt = keys > vstar
    c_gt = jnp.sum(gt.astype(jnp.int32), axis=1, keepdims=True)
    need = k - c_gt
    eq = keys == vstar
    lane = jax.lax.broadcasted_iota(jnp.int32, (B, APAD), 1)

    lo2 = jnp.zeros_like(k)
    hi2 = jnp.full_like(k, APAD)

    def body2(_, lohi):
        lo2, hi2 = lohi
        mid = (lo2 + hi2) >> 1
        gcnt = jnp.sum((eq & (lane < mid)).astype(jnp.int32),
                       axis=1, keepdims=True)
        geq = gcnt >= need
        return (jnp.where(geq, lo2, mid + 1), jnp.where(geq, mid, hi2))

    tstar, _ = jax.lax.fori_loop(0, 14, body2, (lo2, hi2))

    sel = gt | (eq & (lane < tstar))
    neg_ref[...] = jnp.zeros_like(neg_ref) + jnp.sum(jnp.where(sel, ce, 0.0))


def kernel(pos_indicator, predicts, gts):
    pos = pos_indicator.astype(jnp.int32)
    pos = jnp.pad(pos, ((0, 0), (0, APAD - A)))
    posr = pos.reshape(B * NBLK, 1, AB)

    keys, cev, locsum, posce, n = pl.pallas_call(
        _stage_a_body,
        grid=(B, NBLK),
        in_specs=[
            pl.BlockSpec((1, 1, AB), lambda b, a: (b * NBLK + a, 0, 0)),
            pl.BlockSpec((1, AB, C), lambda b, a: (b, a, 0)),
            pl.BlockSpec((1, AB, C), lambda b, a: (b, a, 0)),
        ],
        out_specs=[
            pl.BlockSpec((1, 1, AB), lambda b, a: (b * NBLK + a, 0, 0)),
            pl.BlockSpec((1, 1, AB), lambda b, a: (b * NBLK + a, 0, 0)),
            pl.BlockSpec((1, 1), lambda b, a: (0, 0)),
            pl.BlockSpec((1, 1), lambda b, a: (0, 0)),
            pl.BlockSpec((1, 1), lambda b, a: (0, 0)),
        ],
        out_shape=[
            jax.ShapeDtypeStruct((B * NBLK, 1, AB), jnp.int32),
            jax.ShapeDtypeStruct((B * NBLK, 1, AB), jnp.float32),
            jax.ShapeDtypeStruct((1, 1), jnp.float32),
            jax.ShapeDtypeStruct((1, 1), jnp.float32),
            jax.ShapeDtypeStruct((1, 1), jnp.float32),
        ],
    )(posr, predicts, gts)

    keys2 = keys.reshape(B, APAD)
    ce2 = cev.reshape(B, APAD)

    negsum = pl.pallas_call(
        _stage_b_body,
        in_specs=[
            pl.BlockSpec((B, APAD), lambda: (0, 0)),
            pl.BlockSpec((B, APAD), lambda: (0, 0)),
        ],
        out_specs=pl.BlockSpec((1, 1), lambda: (0, 0)),
        out_shape=jax.ShapeDtypeStruct((1, 1), jnp.float32),
    )(keys2, ce2)

    nn = n[0, 0]
    conf_loss = (posce[0, 0] + negsum[0, 0]) / nn
    loc_loss = locsum[0, 0] / nn
    return (conf_loss, loc_loss)
